# baseline (speedup 1.0000x reference)
import functools

import jax
import jax.numpy as jnp
from jax import lax
from jax.experimental import pallas as pl
from jax.experimental.pallas import tpu as pltpu
from jax.experimental.pallas import tpu_sc as plsc

N = 10000
NP = 10240
H = 128
NE = 4
T = 5
E = 320000
NCLS = 2

NWORK = 32
CHUNK = 128
NCH = 80
EPW = NCH * CHUNK
EPAD = NWORK * EPW
ACC_ROWS = NP + 1024

BLK = 1024
NBLK = NP // BLK


def _mm_bias_body(x_ref, w_ref, b_ref, o_ref):
    o_ref[...] = jnp.dot(x_ref[...], w_ref[...],
                         preferred_element_type=jnp.float32) + b_ref[...]


def _emit_pre(h, wet_ref, bet_ref, whh_ref, bhh_ref, tx_ref, gh_ref):
    hb = h.astype(jnp.bfloat16)
    for e in range(NE):
        tx_ref[e] = (jnp.dot(hb, wet_ref[e], preferred_element_type=jnp.float32)
                     + bet_ref[:, e * H:(e + 1) * H])
    gh_ref[...] = jnp.dot(hb, whh_ref[...],
                          preferred_element_type=jnp.float32) + bhh_ref[...]


def _init_body(x_ref, wred_ref, bred_ref, wet_ref, bet_ref, whh_ref, bhh_ref,
               h0_ref, tx_ref, gh_ref):
    h0 = jnp.dot(x_ref[...].astype(jnp.bfloat16), wred_ref[...],
                 preferred_element_type=jnp.float32) + bred_ref[...]
    h0_ref[...] = h0
    _emit_pre(h0, wet_ref, bet_ref, whh_ref, bhh_ref, tx_ref, gh_ref)


def _emit_gru(a_ref, gh_ref, h_ref, wih_ref, bih_ref):
    a = a_ref[0] + a_ref[1]
    gi = jnp.dot(a.astype(jnp.bfloat16), wih_ref[...],
                 preferred_element_type=jnp.float32) + bih_ref[...]
    gh = gh_ref[...]
    h = h_ref[...]
    r = jax.nn.sigmoid(gi[:, 0:H] + gh[:, 0:H])
    z = jax.nn.sigmoid(gi[:, H:2 * H] + gh[:, H:2 * H])
    n = jnp.tanh(gi[:, 2 * H:3 * H] + r * gh[:, 2 * H:3 * H])
    return (1.0 - z) * n + z * h


def _fused_step_body(a_ref, gh_ref, h_ref, wih_ref, bih_ref, wet_ref, bet_ref,
                     whh_ref, bhh_ref, hn_ref, tx_ref, ghn_ref):
    hn = _emit_gru(a_ref, gh_ref, h_ref, wih_ref, bih_ref)
    hn_ref[...] = hn
    _emit_pre(hn, wet_ref, bet_ref, whh_ref, bhh_ref, tx_ref, ghn_ref)


def _gru_body(a_ref, gh_ref, h_ref, wih_ref, bih_ref, o_ref):
    o_ref[...] = _emit_gru(a_ref, gh_ref, h_ref, wih_ref, bih_ref)


def _pool_body(h_ref, h0_ref, wgh_ref, wgh0_ref, bg_ref, woh_ref, woh0_ref,
               bo_ref, logp_ref, pred_ref):
    h = h_ref[...]
    h0 = h0_ref[...]
    g = (jnp.sum(h * wgh_ref[...], axis=1, keepdims=True)
         + jnp.sum(h0 * wgh0_ref[...], axis=1, keepdims=True) + bg_ref[0, 0])
    rows = lax.broadcasted_iota(jnp.int32, (NP, 1), 0)
    valid = rows < N
    g = jnp.where(valid, g, -jnp.inf)
    m = jnp.max(g)
    e = jnp.where(valid, jnp.exp(g - m), 0.0)
    ssum = jnp.sum(e)
    rh = jnp.sum(e * h, axis=0, keepdims=True) / ssum
    rh0 = jnp.sum(e * h0, axis=0, keepdims=True) / ssum
    logits = (jnp.dot(rh, woh_ref[...], preferred_element_type=jnp.float32)
              + jnp.dot(rh0, woh0_ref[...], preferred_element_type=jnp.float32)
              + bo_ref[...])
    lanes = lax.broadcasted_iota(jnp.int32, (1, 128), 1)
    lmask = lanes < NCLS
    lm = jnp.where(lmask, logits, -jnp.inf)
    m2 = jnp.max(lm)
    lse = m2 + jnp.log(jnp.sum(jnp.where(lmask, jnp.exp(lm - m2), 0.0)))
    logp_ref[...] = jnp.where(lmask, logits - lse, 0.0)
    pred = jnp.where(logits[0, 1] > logits[0, 0], 1, 0).astype(jnp.int32)
    pred_ref[...] = jnp.where(lanes == 0, pred, 0)


def _sc_body(tx_hbm, idx_hbm, out_hbm, ibuf, rows0, rows1, acc,
             sem0, sem1, isem0, isem1, isem2, isem3):
    c = lax.axis_index("c")
    s = lax.axis_index("s")
    w = c * 16 + s
    rows = (rows0, rows1)
    sems = (sem0, sem1)
    isems = (isem0, isem1, isem2, isem3)
    base = w * NCH

    zero16 = jnp.zeros((16,), jnp.float32)

    def _zrow(i, carry):
        for j in range(8):
            rows0[i, pl.ds(j * 16, 16)] = zero16
        return carry

    lax.fori_loop(0, CHUNK, _zrow, 0)
    for k in range(640 // CHUNK):
        pltpu.sync_copy(rows0, acc.at[pl.ds(s * 640 + k * CHUNK, CHUNK)])

    pltpu.sync_copy(idx_hbm.at[base + 0], ibuf.at[0])
    pltpu.sync_copy(idx_hbm.at[base + 1], ibuf.at[1])
    pltpu.async_copy(idx_hbm.at[base + 2], ibuf.at[2], isem2)
    pltpu.async_copy(idx_hbm.at[base + 3], ibuf.at[3], isem3)
    pltpu.async_copy(tx_hbm.at[ibuf.at[0, 0]], rows0, sem0)
    pltpu.async_copy(tx_hbm.at[ibuf.at[1, 0]], rows1, sem1)
    plsc.subcore_barrier()

    def _step(j, jj):
        q, r = j % 4, j % 2
        if j + 2 < NCH:
            pltpu.make_async_copy(idx_hbm.at[base], ibuf.at[(j + 2) % 4],
                                  isems[(j + 2) % 4]).wait()
        pltpu.make_async_copy(tx_hbm.at[pl.ds(0, CHUNK)], rows[r], sems[r]).wait()
        pltpu.sync_copy(rows[r], acc.at[ibuf.at[q, 1]], add=True)
        if j + 2 < NCH:
            pltpu.async_copy(tx_hbm.at[ibuf.at[(j + 2) % 4, 0]], rows[r], sems[r])
        if j + 4 < NCH:
            pltpu.async_copy(idx_hbm.at[jj + 4], ibuf.at[q], isems[q])

    def _quad(i, carry):
        j0 = i * 4
        for b in range(4):
            _step(b, base + j0 + b)
        return carry

    lax.fori_loop(0, (NCH - 8) // 4, _quad, 0)
    for j in range(NCH - 8, NCH):
        _step(j, base + j)
    plsc.subcore_barrier()
    for k in range(640 // CHUNK):
        r0 = s * 640 + k * CHUNK
        pltpu.sync_copy(acc.at[pl.ds(r0, CHUNK)], out_hbm.at[c, pl.ds(r0, CHUNK)])


@functools.cache
def _sc_edge_step():
    mesh = plsc.VectorSubcoreMesh(core_axis_name="c", subcore_axis_name="s",
                                  num_cores=2, num_subcores=16)
    return pl.kernel(
        _sc_body,
        out_type=jax.ShapeDtypeStruct((2, NP, H), jnp.float32),
        mesh=mesh,
        scratch_types=[
            pltpu.VMEM((4, 2, CHUNK), jnp.int32),
            pltpu.VMEM((CHUNK, H), jnp.float32),
            pltpu.VMEM((CHUNK, H), jnp.float32),
            pltpu.VMEM_SHARED((ACC_ROWS, H), jnp.float32),
            pltpu.SemaphoreType.DMA,
            pltpu.SemaphoreType.DMA,
            pltpu.SemaphoreType.DMA,
            pltpu.SemaphoreType.DMA,
            pltpu.SemaphoreType.DMA,
            pltpu.SemaphoreType.DMA,
        ],
    )


def _init_step(x, W_red, b_red2, W_et, b_et2, W_hh, b_hh2):
    return pl.pallas_call(
        _init_body,
        grid=(NBLK,),
        in_specs=[pl.BlockSpec((BLK, H), lambda i: (i, 0)),
                  pl.BlockSpec((H, H), lambda i: (0, 0)),
                  pl.BlockSpec((1, H), lambda i: (0, 0)),
                  pl.BlockSpec((NE, H, H), lambda i: (0, 0, 0)),
                  pl.BlockSpec((1, NE * H), lambda i: (0, 0)),
                  pl.BlockSpec((H, 3 * H), lambda i: (0, 0)),
                  pl.BlockSpec((1, 3 * H), lambda i: (0, 0))],
        out_specs=[pl.BlockSpec((BLK, H), lambda i: (i, 0)),
                   pl.BlockSpec((NE, BLK, H), lambda i: (0, i, 0)),
                   pl.BlockSpec((BLK, 3 * H), lambda i: (i, 0))],
        out_shape=[jax.ShapeDtypeStruct((NP, H), jnp.float32),
                   jax.ShapeDtypeStruct((NE, NP, H), jnp.float32),
                   jax.ShapeDtypeStruct((NP, 3 * H), jnp.float32)],
    )(x, W_red, b_red2, W_et, b_et2, W_hh, b_hh2)


def _fused_step(a2, gh, h, W_ih, b_ih2, W_et, b_et2, W_hh, b_hh2):
    return pl.pallas_call(
        _fused_step_body,
        grid=(NBLK,),
        in_specs=[pl.BlockSpec((2, BLK, H), lambda i: (0, i, 0)),
                  pl.BlockSpec((BLK, 3 * H), lambda i: (i, 0)),
                  pl.BlockSpec((BLK, H), lambda i: (i, 0)),
                  pl.BlockSpec((H, 3 * H), lambda i: (0, 0)),
                  pl.BlockSpec((1, 3 * H), lambda i: (0, 0)),
                  pl.BlockSpec((NE, H, H), lambda i: (0, 0, 0)),
                  pl.BlockSpec((1, NE * H), lambda i: (0, 0)),
                  pl.BlockSpec((H, 3 * H), lambda i: (0, 0)),
                  pl.BlockSpec((1, 3 * H), lambda i: (0, 0))],
        out_specs=[pl.BlockSpec((BLK, H), lambda i: (i, 0)),
                   pl.BlockSpec((NE, BLK, H), lambda i: (0, i, 0)),
                   pl.BlockSpec((BLK, 3 * H), lambda i: (i, 0))],
        out_shape=[jax.ShapeDtypeStruct((NP, H), jnp.float32),
                   jax.ShapeDtypeStruct((NE, NP, H), jnp.float32),
                   jax.ShapeDtypeStruct((NP, 3 * H), jnp.float32)],
    )(a2, gh, h, W_ih, b_ih2, W_et, b_et2, W_hh, b_hh2)


def _gru_step(a2, gh, h, W_ih, b_ih2):
    return pl.pallas_call(
        _gru_body,
        grid=(NBLK,),
        in_specs=[pl.BlockSpec((2, BLK, H), lambda i: (0, i, 0)),
                  pl.BlockSpec((BLK, 3 * H), lambda i: (i, 0)),
                  pl.BlockSpec((BLK, H), lambda i: (i, 0)),
                  pl.BlockSpec((H, 3 * H), lambda i: (0, 0)),
                  pl.BlockSpec((1, 3 * H), lambda i: (0, 0))],
        out_specs=pl.BlockSpec((BLK, H), lambda i: (i, 0)),
        out_shape=jax.ShapeDtypeStruct((NP, H), jnp.float32),
    )(a2, gh, h, W_ih, b_ih2)


def _pool(h, h0, wgh, wgh0, bg, woh, woh0, bo):
    return pl.pallas_call(
        _pool_body,
        in_specs=[pl.BlockSpec((NP, H), lambda: (0, 0)),
                  pl.BlockSpec((NP, H), lambda: (0, 0)),
                  pl.BlockSpec((1, H), lambda: (0, 0)),
                  pl.BlockSpec((1, H), lambda: (0, 0)),
                  pl.BlockSpec((1, 1), lambda: (0, 0)),
                  pl.BlockSpec((H, 128), lambda: (0, 0)),
                  pl.BlockSpec((H, 128), lambda: (0, 0)),
                  pl.BlockSpec((1, 128), lambda: (0, 0))],
        out_specs=[pl.BlockSpec((1, 128), lambda: (0, 0)),
                   pl.BlockSpec((1, 128), lambda: (0, 0))],
        out_shape=[jax.ShapeDtypeStruct((1, 128), jnp.float32),
                   jax.ShapeDtypeStruct((1, 128), jnp.int32)],
    )(h, h0, wgh, wgh0, bg, woh, woh0, bo)


def kernel(annotation, edge_index, etypes, labels, W_red, b_red, W_et, b_et,
           W_ih, b_ih, W_hh, b_hh, W_gate, b_gate, W_out, b_out):
    src = edge_index[0].astype(jnp.int32)
    dst = edge_index[1].astype(jnp.int32)
    et = etypes.astype(jnp.int32)
    gidx = et * NP + src
    npad = EPAD - E
    pad_g = jnp.arange(npad, dtype=jnp.int32) % 1024
    pad_d = NP + jnp.arange(npad, dtype=jnp.int32) % 1024
    gidx2 = jnp.concatenate([gidx, pad_g]).reshape(EPAD // CHUNK, CHUNK)
    dst2 = jnp.concatenate([dst, pad_d]).reshape(EPAD // CHUNK, CHUNK)
    idx_comb = jnp.stack([gidx2, dst2], axis=1)

    ann_p = jnp.pad(annotation.astype(jnp.float32), ((0, NP - N), (0, 0)))
    b_et2 = b_et.reshape(1, NE * H)
    b_hh2 = b_hh.reshape(1, 3 * H)
    b_ih2 = b_ih.reshape(1, 3 * H)
    wgh = W_gate[:H].reshape(1, H)
    wgh0 = W_gate[H:].reshape(1, H)
    bg = b_gate.reshape(1, 1)
    woh = jnp.zeros((H, 128), jnp.float32).at[:, :NCLS].set(W_out[:H])
    woh0 = jnp.zeros((H, 128), jnp.float32).at[:, :NCLS].set(W_out[H:])
    bo = jnp.zeros((1, 128), jnp.float32).at[0, :NCLS].set(b_out)

    W_red_b = W_red.astype(jnp.bfloat16)
    W_et_b = W_et.astype(jnp.bfloat16)
    W_hh_b = W_hh.astype(jnp.bfloat16)
    W_ih_b = W_ih.astype(jnp.bfloat16)

    h0, tx, gh = _init_step(ann_p, W_red_b, b_red.reshape(1, H), W_et_b, b_et2,
                            W_hh_b, b_hh2)
    h = h0
    for _ in range(T - 1):
        a2 = _sc_edge_step()(tx.reshape(NE * NP, H), idx_comb)
        h, tx, gh = _fused_step(a2, gh, h, W_ih_b, b_ih2, W_et_b, b_et2,
                                W_hh_b, b_hh2)
    a2 = _sc_edge_step()(tx.reshape(NE * NP, H), idx_comb)
    h = _gru_step(a2, gh, h, W_ih_b, b_ih2)

    logp_full, pred_full = _pool(h, h0, wgh, wgh0, bg, woh, woh0, bo)
    logp2 = logp_full[0, :NCLS]
    loss = -jnp.take(logp2, labels.astype(jnp.int32)).mean()
    preds = pred_full[0, :1]
    return (loss, preds)


# speedup vs baseline: 1.1048x; 1.1048x over previous
import functools

import jax
import jax.numpy as jnp
from jax import lax
from jax.experimental import pallas as pl
from jax.experimental.pallas import tpu as pltpu
from jax.experimental.pallas import tpu_sc as plsc

N = 10000
NP = 10240
H = 128
NE = 4
T = 5
E = 320000
NCLS = 2

NWORK = 32
CHUNK = 128
NCH = 80
EPW = NCH * CHUNK
EPAD = NWORK * EPW
ACC_ROWS = N + 40

BLK = 1024
NBLK = NP // BLK


def _mm_bias_body(x_ref, w_ref, b_ref, o_ref):
    o_ref[...] = jnp.dot(x_ref[...], w_ref[...],
                         preferred_element_type=jnp.float32) + b_ref[...]


def _emit_pre(h, wet_ref, bet_ref, whh_ref, bhh_ref, tx_ref, gh_ref):
    for e in range(NE):
        tx_ref[e] = (jnp.dot(h, wet_ref[e], preferred_element_type=jnp.float32)
                     + bet_ref[:, e * H:(e + 1) * H])
    gh_ref[...] = jnp.dot(h, whh_ref[...],
                          preferred_element_type=jnp.float32) + bhh_ref[...]


def _init_body(x_ref, wred_ref, bred_ref, wet_ref, bet_ref, whh_ref, bhh_ref,
               h0_ref, tx_ref, gh_ref):
    h0 = jnp.dot(x_ref[...], wred_ref[...],
                 preferred_element_type=jnp.float32) + bred_ref[...]
    h0_ref[...] = h0
    _emit_pre(h0, wet_ref, bet_ref, whh_ref, bhh_ref, tx_ref, gh_ref)


def _emit_gru(a_ref, gh_ref, h_ref, wih_ref, bih_ref):
    a = a_ref[0] + a_ref[1]
    gi = jnp.dot(a, wih_ref[...], preferred_element_type=jnp.float32) + bih_ref[...]
    gh = gh_ref[...]
    h = h_ref[...]
    r = jax.nn.sigmoid(gi[:, 0:H] + gh[:, 0:H])
    z = jax.nn.sigmoid(gi[:, H:2 * H] + gh[:, H:2 * H])
    n = jnp.tanh(gi[:, 2 * H:3 * H] + r * gh[:, 2 * H:3 * H])
    return (1.0 - z) * n + z * h


def _fused_step_body(a_ref, gh_ref, h_ref, wih_ref, bih_ref, wet_ref, bet_ref,
                     whh_ref, bhh_ref, hn_ref, tx_ref, ghn_ref):
    hn = _emit_gru(a_ref, gh_ref, h_ref, wih_ref, bih_ref)
    hn_ref[...] = hn
    _emit_pre(hn, wet_ref, bet_ref, whh_ref, bhh_ref, tx_ref, ghn_ref)


def _gru_body(a_ref, gh_ref, h_ref, wih_ref, bih_ref, o_ref):
    o_ref[...] = _emit_gru(a_ref, gh_ref, h_ref, wih_ref, bih_ref)


def _pool_body(h_ref, h0_ref, wgh_ref, wgh0_ref, bg_ref, woh_ref, woh0_ref,
               bo_ref, logp_ref, pred_ref):
    h = h_ref[...]
    h0 = h0_ref[...]
    g = (jnp.sum(h * wgh_ref[...], axis=1, keepdims=True)
         + jnp.sum(h0 * wgh0_ref[...], axis=1, keepdims=True) + bg_ref[0, 0])
    rows = lax.broadcasted_iota(jnp.int32, (NP, 1), 0)
    valid = rows < N
    g = jnp.where(valid, g, -jnp.inf)
    m = jnp.max(g)
    e = jnp.where(valid, jnp.exp(g - m), 0.0)
    ssum = jnp.sum(e)
    rh = jnp.sum(e * h, axis=0, keepdims=True) / ssum
    rh0 = jnp.sum(e * h0, axis=0, keepdims=True) / ssum
    logits = (jnp.dot(rh, woh_ref[...], preferred_element_type=jnp.float32)
              + jnp.dot(rh0, woh0_ref[...], preferred_element_type=jnp.float32)
              + bo_ref[...])
    lanes = lax.broadcasted_iota(jnp.int32, (1, 128), 1)
    lmask = lanes < NCLS
    lm = jnp.where(lmask, logits, -jnp.inf)
    m2 = jnp.max(lm)
    lse = m2 + jnp.log(jnp.sum(jnp.where(lmask, jnp.exp(lm - m2), 0.0)))
    logp_ref[...] = jnp.where(lmask, logits - lse, 0.0)
    pred = jnp.where(logits[0, 1] > logits[0, 0], 1, 0).astype(jnp.int32)
    pred_ref[...] = jnp.where(lanes == 0, pred, 0)


def _sc_body(tx_hbm, idx_hbm, out_hbm, ibuf, rows0, rows1, rows2, acc,
             sem0, sem1, sem2, isem0, isem1, isem2, isem3, isem4, isem5):
    c = lax.axis_index("c")
    s = lax.axis_index("s")
    w = c * 16 + s
    rows = (rows0, rows1, rows2)
    sems = (sem0, sem1, sem2)
    isems = (isem0, isem1, isem2, isem3, isem4, isem5)
    base = w * NCH

    zero16 = jnp.zeros((16,), jnp.float32)

    def _zrow(i, carry):
        for j in range(8):
            rows0[i, pl.ds(j * 16, 16)] = zero16
        return carry

    lax.fori_loop(0, CHUNK, _zrow, 0)
    for k in range(4):
        pltpu.sync_copy(rows0, acc.at[pl.ds(s * 627 + k * CHUNK, CHUNK)])

    @pl.when(s == 15)
    def _zero_tail():
        pltpu.sync_copy(rows0.at[pl.ds(0, 123)], acc.at[pl.ds(9917, 123)])

    @pl.when(s < 15)
    def _zero_mid():
        pltpu.sync_copy(rows0.at[pl.ds(0, 115)],
                        acc.at[pl.ds(s * 627 + 512, 115)])

    for q in range(3):
        pltpu.sync_copy(idx_hbm.at[base + q], ibuf.at[q])
    for q in range(3, 6):
        pltpu.async_copy(idx_hbm.at[base + q], ibuf.at[q], isems[q])
    for q in range(3):
        pltpu.async_copy(tx_hbm.at[ibuf.at[q, 0]], rows[q], sems[q])
    plsc.subcore_barrier()

    def _step(j, jj):
        q3, q6 = j % 3, j % 6
        if j + 3 < NCH:
            pltpu.make_async_copy(idx_hbm.at[base], ibuf.at[(j + 3) % 6],
                                  isems[(j + 3) % 6]).wait()
        pltpu.make_async_copy(tx_hbm.at[pl.ds(0, CHUNK)], rows[q3], sems[q3]).wait()
        pltpu.sync_copy(rows[q3], acc.at[ibuf.at[q6, 1]], add=True)
        if j + 3 < NCH:
            pltpu.async_copy(tx_hbm.at[ibuf.at[(j + 3) % 6, 0]], rows[q3], sems[q3])
        if j + 6 < NCH:
            pltpu.async_copy(idx_hbm.at[jj + 6], ibuf.at[q6], isems[q6])

    def _hex(i, carry):
        j0 = i * 6
        for b in range(6):
            _step(b, base + j0 + b)
        return carry

    NMAIN = ((NCH - 12) // 6) * 6
    lax.fori_loop(0, NMAIN // 6, _hex, 0)
    for j in range(NMAIN, NCH):
        _step(j, base + j)
    plsc.subcore_barrier()

    @pl.when(s < 15)
    def _write_mid():
        for k in range(5):
            r0 = s * 640 + k * CHUNK
            pltpu.sync_copy(acc.at[pl.ds(r0, CHUNK)], out_hbm.at[c, pl.ds(r0, CHUNK)])

    @pl.when(s == 15)
    def _write_tail():
        for dst_off, src_off, n in ((9600, 9600, 128), (9728, 9728, 128),
                                    (9856, 9856, 128), (9984, 9984, 16),
                                    (10000, 0, 128), (10128, 0, 112)):
            pltpu.sync_copy(acc.at[pl.ds(src_off, n)],
                            out_hbm.at[c, pl.ds(dst_off, n)])


@functools.cache
def _sc_edge_step():
    mesh = plsc.VectorSubcoreMesh(core_axis_name="c", subcore_axis_name="s",
                                  num_cores=2, num_subcores=16)
    return pl.kernel(
        _sc_body,
        out_type=jax.ShapeDtypeStruct((2, NP, H), jnp.float32),
        mesh=mesh,
        scratch_types=(
            [pltpu.VMEM((6, 2, CHUNK), jnp.int32)]
            + [pltpu.VMEM((CHUNK, H), jnp.float32) for _ in range(3)]
            + [pltpu.VMEM_SHARED((ACC_ROWS, H), jnp.float32)]
            + [pltpu.SemaphoreType.DMA] * 9
        ),
    )


def _init_step(x, W_red, b_red2, W_et, b_et2, W_hh, b_hh2):
    return pl.pallas_call(
        _init_body,
        grid=(NBLK,),
        in_specs=[pl.BlockSpec((BLK, H), lambda i: (i, 0)),
                  pl.BlockSpec((H, H), lambda i: (0, 0)),
                  pl.BlockSpec((1, H), lambda i: (0, 0)),
                  pl.BlockSpec((NE, H, H), lambda i: (0, 0, 0)),
                  pl.BlockSpec((1, NE * H), lambda i: (0, 0)),
                  pl.BlockSpec((H, 3 * H), lambda i: (0, 0)),
                  pl.BlockSpec((1, 3 * H), lambda i: (0, 0))],
        out_specs=[pl.BlockSpec((BLK, H), lambda i: (i, 0)),
                   pl.BlockSpec((NE, BLK, H), lambda i: (0, i, 0)),
                   pl.BlockSpec((BLK, 3 * H), lambda i: (i, 0))],
        out_shape=[jax.ShapeDtypeStruct((NP, H), jnp.float32),
                   jax.ShapeDtypeStruct((NE, NP, H), jnp.float32),
                   jax.ShapeDtypeStruct((NP, 3 * H), jnp.float32)],
    )(x, W_red, b_red2, W_et, b_et2, W_hh, b_hh2)


def _fused_step(a2, gh, h, W_ih, b_ih2, W_et, b_et2, W_hh, b_hh2):
    return pl.pallas_call(
        _fused_step_body,
        grid=(NBLK,),
        in_specs=[pl.BlockSpec((2, BLK, H), lambda i: (0, i, 0)),
                  pl.BlockSpec((BLK, 3 * H), lambda i: (i, 0)),
                  pl.BlockSpec((BLK, H), lambda i: (i, 0)),
                  pl.BlockSpec((H, 3 * H), lambda i: (0, 0)),
                  pl.BlockSpec((1, 3 * H), lambda i: (0, 0)),
                  pl.BlockSpec((NE, H, H), lambda i: (0, 0, 0)),
                  pl.BlockSpec((1, NE * H), lambda i: (0, 0)),
                  pl.BlockSpec((H, 3 * H), lambda i: (0, 0)),
                  pl.BlockSpec((1, 3 * H), lambda i: (0, 0))],
        out_specs=[pl.BlockSpec((BLK, H), lambda i: (i, 0)),
                   pl.BlockSpec((NE, BLK, H), lambda i: (0, i, 0)),
                   pl.BlockSpec((BLK, 3 * H), lambda i: (i, 0))],
        out_shape=[jax.ShapeDtypeStruct((NP, H), jnp.float32),
                   jax.ShapeDtypeStruct((NE, NP, H), jnp.float32),
                   jax.ShapeDtypeStruct((NP, 3 * H), jnp.float32)],
    )(a2, gh, h, W_ih, b_ih2, W_et, b_et2, W_hh, b_hh2)


def _gru_step(a2, gh, h, W_ih, b_ih2):
    return pl.pallas_call(
        _gru_body,
        grid=(NBLK,),
        in_specs=[pl.BlockSpec((2, BLK, H), lambda i: (0, i, 0)),
                  pl.BlockSpec((BLK, 3 * H), lambda i: (i, 0)),
                  pl.BlockSpec((BLK, H), lambda i: (i, 0)),
                  pl.BlockSpec((H, 3 * H), lambda i: (0, 0)),
                  pl.BlockSpec((1, 3 * H), lambda i: (0, 0))],
        out_specs=pl.BlockSpec((BLK, H), lambda i: (i, 0)),
        out_shape=jax.ShapeDtypeStruct((NP, H), jnp.float32),
    )(a2, gh, h, W_ih, b_ih2)


def _pool(h, h0, wgh, wgh0, bg, woh, woh0, bo):
    return pl.pallas_call(
        _pool_body,
        in_specs=[pl.BlockSpec((NP, H), lambda: (0, 0)),
                  pl.BlockSpec((NP, H), lambda: (0, 0)),
                  pl.BlockSpec((1, H), lambda: (0, 0)),
                  pl.BlockSpec((1, H), lambda: (0, 0)),
                  pl.BlockSpec((1, 1), lambda: (0, 0)),
                  pl.BlockSpec((H, 128), lambda: (0, 0)),
                  pl.BlockSpec((H, 128), lambda: (0, 0)),
                  pl.BlockSpec((1, 128), lambda: (0, 0))],
        out_specs=[pl.BlockSpec((1, 128), lambda: (0, 0)),
                   pl.BlockSpec((1, 128), lambda: (0, 0))],
        out_shape=[jax.ShapeDtypeStruct((1, 128), jnp.float32),
                   jax.ShapeDtypeStruct((1, 128), jnp.int32)],
    )(h, h0, wgh, wgh0, bg, woh, woh0, bo)


def kernel(annotation, edge_index, etypes, labels, W_red, b_red, W_et, b_et,
           W_ih, b_ih, W_hh, b_hh, W_gate, b_gate, W_out, b_out):
    src = edge_index[0].astype(jnp.int32)
    dst = edge_index[1].astype(jnp.int32)
    et = etypes.astype(jnp.int32)
    gidx = et * NP + src
    npad = EPAD - E
    pad_g = jnp.arange(npad, dtype=jnp.int32) % 1024
    pad_d = N + jnp.arange(npad, dtype=jnp.int32) % 40
    gidx2 = jnp.concatenate([gidx, pad_g]).reshape(EPAD // CHUNK, CHUNK)
    dst2 = jnp.concatenate([dst, pad_d]).reshape(EPAD // CHUNK, CHUNK)
    idx_comb = jnp.stack([gidx2, dst2], axis=1)

    ann_p = jnp.pad(annotation.astype(jnp.float32), ((0, NP - N), (0, 0)))
    b_et2 = b_et.reshape(1, NE * H)
    b_hh2 = b_hh.reshape(1, 3 * H)
    b_ih2 = b_ih.reshape(1, 3 * H)
    wgh = W_gate[:H].reshape(1, H)
    wgh0 = W_gate[H:].reshape(1, H)
    bg = b_gate.reshape(1, 1)
    woh = jnp.zeros((H, 128), jnp.float32).at[:, :NCLS].set(W_out[:H])
    woh0 = jnp.zeros((H, 128), jnp.float32).at[:, :NCLS].set(W_out[H:])
    bo = jnp.zeros((1, 128), jnp.float32).at[0, :NCLS].set(b_out)

    h0, tx, gh = _init_step(ann_p, W_red, b_red.reshape(1, H), W_et, b_et2,
                            W_hh, b_hh2)
    h = h0
    for _ in range(T - 1):
        a2 = _sc_edge_step()(tx.reshape(NE * NP, H), idx_comb)
        h, tx, gh = _fused_step(a2, gh, h, W_ih, b_ih2, W_et, b_et2, W_hh, b_hh2)
    a2 = _sc_edge_step()(tx.reshape(NE * NP, H), idx_comb)
    h = _gru_step(a2, gh, h, W_ih, b_ih2)

    logp_full, pred_full = _pool(h, h0, wgh, wgh0, bg, woh, woh0, bo)
    logp2 = logp_full[0, :NCLS]
    loss = -jnp.take(logp2, labels.astype(jnp.int32)).mean()
    preds = pred_full[0, :1]
    return (loss, preds)


# speedup vs baseline: 1.1656x; 1.0550x over previous
import functools

import jax
import jax.numpy as jnp
from jax import lax
from jax.experimental import pallas as pl
from jax.experimental.pallas import tpu as pltpu
from jax.experimental.pallas import tpu_sc as plsc

N = 10000
NP = 10240
H = 128
NE = 4
T = 5
E = 320000
NCLS = 2

NWORK = 32
CHUNK = 128
NCH = 80
EPW = NCH * CHUNK
EPAD = NWORK * EPW
ACC_ROWS = N + 40

BLK = 1024
NBLK = NP // BLK


def _mm_bias_body(x_ref, w_ref, b_ref, o_ref):
    o_ref[...] = jnp.dot(x_ref[...], w_ref[...],
                         preferred_element_type=jnp.float32) + b_ref[...]


def _emit_pre(h, wet_ref, bet_ref, tx_ref):
    for e in range(NE):
        tx_ref[e] = (jnp.dot(h, wet_ref[e], preferred_element_type=jnp.float32)
                     + bet_ref[:, e * H:(e + 1) * H])


def _init_body(x_ref, wred_ref, bred_ref, wet_ref, bet_ref, h0_ref, tx_ref):
    h0 = jnp.dot(x_ref[...], wred_ref[...],
                 preferred_element_type=jnp.float32) + bred_ref[...]
    h0_ref[...] = h0
    _emit_pre(h0, wet_ref, bet_ref, tx_ref)


def _emit_gru(a_ref, h_ref, wih_ref, bih_ref, whh_ref, bhh_ref):
    a = a_ref[0] + a_ref[1]
    gi = jnp.dot(a, wih_ref[...], preferred_element_type=jnp.float32) + bih_ref[...]
    h = h_ref[...]
    gh = jnp.dot(h, whh_ref[...], preferred_element_type=jnp.float32) + bhh_ref[...]
    r = jax.nn.sigmoid(gi[:, 0:H] + gh[:, 0:H])
    z = jax.nn.sigmoid(gi[:, H:2 * H] + gh[:, H:2 * H])
    n = jnp.tanh(gi[:, 2 * H:3 * H] + r * gh[:, 2 * H:3 * H])
    return (1.0 - z) * n + z * h


def _fused_step_body(a_ref, h_ref, wih_ref, bih_ref, whh_ref, bhh_ref,
                     wet_ref, bet_ref, hn_ref, tx_ref):
    hn = _emit_gru(a_ref, h_ref, wih_ref, bih_ref, whh_ref, bhh_ref)
    hn_ref[...] = hn
    _emit_pre(hn, wet_ref, bet_ref, tx_ref)


def _gru_body(a_ref, h_ref, wih_ref, bih_ref, whh_ref, bhh_ref, o_ref):
    o_ref[...] = _emit_gru(a_ref, h_ref, wih_ref, bih_ref, whh_ref, bhh_ref)


def _pool_body(h_ref, h0_ref, wgh_ref, wgh0_ref, bg_ref, woh_ref, woh0_ref,
               bo_ref, logp_ref, pred_ref):
    h = h_ref[...]
    h0 = h0_ref[...]
    g = (jnp.sum(h * wgh_ref[...], axis=1, keepdims=True)
         + jnp.sum(h0 * wgh0_ref[...], axis=1, keepdims=True) + bg_ref[0, 0])
    rows = lax.broadcasted_iota(jnp.int32, (NP, 1), 0)
    valid = rows < N
    g = jnp.where(valid, g, -jnp.inf)
    m = jnp.max(g)
    e = jnp.where(valid, jnp.exp(g - m), 0.0)
    ssum = jnp.sum(e)
    rh = jnp.sum(e * h, axis=0, keepdims=True) / ssum
    rh0 = jnp.sum(e * h0, axis=0, keepdims=True) / ssum
    logits = (jnp.dot(rh, woh_ref[...], preferred_element_type=jnp.float32)
              + jnp.dot(rh0, woh0_ref[...], preferred_element_type=jnp.float32)
              + bo_ref[...])
    lanes = lax.broadcasted_iota(jnp.int32, (1, 128), 1)
    lmask = lanes < NCLS
    lm = jnp.where(lmask, logits, -jnp.inf)
    m2 = jnp.max(lm)
    lse = m2 + jnp.log(jnp.sum(jnp.where(lmask, jnp.exp(lm - m2), 0.0)))
    logp_ref[...] = jnp.where(lmask, logits - lse, 0.0)
    pred = jnp.where(logits[0, 1] > logits[0, 0], 1, 0).astype(jnp.int32)
    pred_ref[...] = jnp.where(lanes == 0, pred, 0)


def _sc_body(tx_hbm, idx_hbm, out_hbm, ibuf, rows0, rows1, rows2, acc,
             sem0, sem1, sem2, isem0, isem1, isem2, isem3, isem4, isem5):
    c = lax.axis_index("c")
    s = lax.axis_index("s")
    w = c * 16 + s
    rows = (rows0, rows1, rows2)
    sems = (sem0, sem1, sem2)
    isems = (isem0, isem1, isem2, isem3, isem4, isem5)
    base = w * NCH

    zero16 = jnp.zeros((16,), jnp.float32)

    def _zrow(i, carry):
        for j in range(8):
            rows0[i, pl.ds(j * 16, 16)] = zero16
        return carry

    lax.fori_loop(0, CHUNK, _zrow, 0)
    for k in range(4):
        pltpu.sync_copy(rows0, acc.at[pl.ds(s * 627 + k * CHUNK, CHUNK)])

    @pl.when(s == 15)
    def _zero_tail():
        pltpu.sync_copy(rows0.at[pl.ds(0, 123)], acc.at[pl.ds(9917, 123)])

    @pl.when(s < 15)
    def _zero_mid():
        pltpu.sync_copy(rows0.at[pl.ds(0, 115)],
                        acc.at[pl.ds(s * 627 + 512, 115)])

    for q in range(3):
        pltpu.sync_copy(idx_hbm.at[base + q], ibuf.at[q])
    for q in range(3, 6):
        pltpu.async_copy(idx_hbm.at[base + q], ibuf.at[q], isems[q])
    for q in range(3):
        pltpu.async_copy(tx_hbm.at[ibuf.at[q, 0]], rows[q], sems[q])
    plsc.subcore_barrier()

    def _step(j, jj):
        q3, q6 = j % 3, j % 6
        if j + 3 < NCH:
            pltpu.make_async_copy(idx_hbm.at[base], ibuf.at[(j + 3) % 6],
                                  isems[(j + 3) % 6]).wait()
        pltpu.make_async_copy(tx_hbm.at[pl.ds(0, CHUNK)], rows[q3], sems[q3]).wait()
        pltpu.sync_copy(rows[q3], acc.at[ibuf.at[q6, 1]], add=True)
        if j + 3 < NCH:
            pltpu.async_copy(tx_hbm.at[ibuf.at[(j + 3) % 6, 0]], rows[q3], sems[q3])
        if j + 6 < NCH:
            pltpu.async_copy(idx_hbm.at[jj + 6], ibuf.at[q6], isems[q6])

    def _hex(i, carry):
        j0 = i * 6
        for b in range(6):
            _step(b, base + j0 + b)
        return carry

    NMAIN = ((NCH - 12) // 6) * 6
    lax.fori_loop(0, NMAIN // 6, _hex, 0)
    for j in range(NMAIN, NCH):
        _step(j, base + j)
    plsc.subcore_barrier()

    @pl.when(s < 15)
    def _write_mid():
        for k in range(5):
            r0 = s * 640 + k * CHUNK
            pltpu.sync_copy(acc.at[pl.ds(r0, CHUNK)], out_hbm.at[c, pl.ds(r0, CHUNK)])

    @pl.when(s == 15)
    def _write_tail():
        for dst_off, src_off, n in ((9600, 9600, 128), (9728, 9728, 128),
                                    (9856, 9856, 128), (9984, 9984, 16),
                                    (10000, 0, 128), (10128, 0, 112)):
            pltpu.sync_copy(acc.at[pl.ds(src_off, n)],
                            out_hbm.at[c, pl.ds(dst_off, n)])


@functools.cache
def _sc_edge_step():
    mesh = plsc.VectorSubcoreMesh(core_axis_name="c", subcore_axis_name="s",
                                  num_cores=2, num_subcores=16)
    return pl.kernel(
        _sc_body,
        out_type=jax.ShapeDtypeStruct((2, NP, H), jnp.float32),
        mesh=mesh,
        scratch_types=(
            [pltpu.VMEM((6, 2, CHUNK), jnp.int32)]
            + [pltpu.VMEM((CHUNK, H), jnp.float32) for _ in range(3)]
            + [pltpu.VMEM_SHARED((ACC_ROWS, H), jnp.float32)]
            + [pltpu.SemaphoreType.DMA] * 9
        ),
    )


def _init_step(x, W_red, b_red2, W_et, b_et2):
    return pl.pallas_call(
        _init_body,
        grid=(NBLK,),
        in_specs=[pl.BlockSpec((BLK, H), lambda i: (i, 0)),
                  pl.BlockSpec((H, H), lambda i: (0, 0)),
                  pl.BlockSpec((1, H), lambda i: (0, 0)),
                  pl.BlockSpec((NE, H, H), lambda i: (0, 0, 0)),
                  pl.BlockSpec((1, NE * H), lambda i: (0, 0))],
        out_specs=[pl.BlockSpec((BLK, H), lambda i: (i, 0)),
                   pl.BlockSpec((NE, BLK, H), lambda i: (0, i, 0))],
        out_shape=[jax.ShapeDtypeStruct((NP, H), jnp.float32),
                   jax.ShapeDtypeStruct((NE, NP, H), jnp.float32)],
    )(x, W_red, b_red2, W_et, b_et2)


def _fused_step(a2, h, W_ih, b_ih2, W_hh, b_hh2, W_et, b_et2):
    return pl.pallas_call(
        _fused_step_body,
        grid=(NBLK,),
        in_specs=[pl.BlockSpec((2, BLK, H), lambda i: (0, i, 0)),
                  pl.BlockSpec((BLK, H), lambda i: (i, 0)),
                  pl.BlockSpec((H, 3 * H), lambda i: (0, 0)),
                  pl.BlockSpec((1, 3 * H), lambda i: (0, 0)),
                  pl.BlockSpec((H, 3 * H), lambda i: (0, 0)),
                  pl.BlockSpec((1, 3 * H), lambda i: (0, 0)),
                  pl.BlockSpec((NE, H, H), lambda i: (0, 0, 0)),
                  pl.BlockSpec((1, NE * H), lambda i: (0, 0))],
        out_specs=[pl.BlockSpec((BLK, H), lambda i: (i, 0)),
                   pl.BlockSpec((NE, BLK, H), lambda i: (0, i, 0))],
        out_shape=[jax.ShapeDtypeStruct((NP, H), jnp.float32),
                   jax.ShapeDtypeStruct((NE, NP, H), jnp.float32)],
    )(a2, h, W_ih, b_ih2, W_hh, b_hh2, W_et, b_et2)


def _gru_step(a2, h, W_ih, b_ih2, W_hh, b_hh2):
    return pl.pallas_call(
        _gru_body,
        grid=(NBLK,),
        in_specs=[pl.BlockSpec((2, BLK, H), lambda i: (0, i, 0)),
                  pl.BlockSpec((BLK, H), lambda i: (i, 0)),
                  pl.BlockSpec((H, 3 * H), lambda i: (0, 0)),
                  pl.BlockSpec((1, 3 * H), lambda i: (0, 0)),
                  pl.BlockSpec((H, 3 * H), lambda i: (0, 0)),
                  pl.BlockSpec((1, 3 * H), lambda i: (0, 0))],
        out_specs=pl.BlockSpec((BLK, H), lambda i: (i, 0)),
        out_shape=jax.ShapeDtypeStruct((NP, H), jnp.float32),
    )(a2, h, W_ih, b_ih2, W_hh, b_hh2)


def _pool(h, h0, wgh, wgh0, bg, woh, woh0, bo):
    return pl.pallas_call(
        _pool_body,
        in_specs=[pl.BlockSpec((NP, H), lambda: (0, 0)),
                  pl.BlockSpec((NP, H), lambda: (0, 0)),
                  pl.BlockSpec((1, H), lambda: (0, 0)),
                  pl.BlockSpec((1, H), lambda: (0, 0)),
                  pl.BlockSpec((1, 1), lambda: (0, 0)),
                  pl.BlockSpec((H, 128), lambda: (0, 0)),
                  pl.BlockSpec((H, 128), lambda: (0, 0)),
                  pl.BlockSpec((1, 128), lambda: (0, 0))],
        out_specs=[pl.BlockSpec((1, 128), lambda: (0, 0)),
                   pl.BlockSpec((1, 128), lambda: (0, 0))],
        out_shape=[jax.ShapeDtypeStruct((1, 128), jnp.float32),
                   jax.ShapeDtypeStruct((1, 128), jnp.int32)],
    )(h, h0, wgh, wgh0, bg, woh, woh0, bo)


def kernel(annotation, edge_index, etypes, labels, W_red, b_red, W_et, b_et,
           W_ih, b_ih, W_hh, b_hh, W_gate, b_gate, W_out, b_out):
    src = edge_index[0].astype(jnp.int32)
    dst = edge_index[1].astype(jnp.int32)
    et = etypes.astype(jnp.int32)
    gidx = et * NP + src
    npad = EPAD - E
    pad_g = jnp.arange(npad, dtype=jnp.int32) % 1024
    pad_d = N + jnp.arange(npad, dtype=jnp.int32) % 40
    gidx2 = jnp.concatenate([gidx, pad_g]).reshape(EPAD // CHUNK, CHUNK)
    dst2 = jnp.concatenate([dst, pad_d]).reshape(EPAD // CHUNK, CHUNK)
    idx_comb = jnp.stack([gidx2, dst2], axis=1)

    ann_p = jnp.pad(annotation.astype(jnp.float32), ((0, NP - N), (0, 0)))
    b_et2 = b_et.reshape(1, NE * H)
    b_hh2 = b_hh.reshape(1, 3 * H)
    b_ih2 = b_ih.reshape(1, 3 * H)
    wgh = W_gate[:H].reshape(1, H)
    wgh0 = W_gate[H:].reshape(1, H)
    bg = b_gate.reshape(1, 1)
    woh = jnp.zeros((H, 128), jnp.float32).at[:, :NCLS].set(W_out[:H])
    woh0 = jnp.zeros((H, 128), jnp.float32).at[:, :NCLS].set(W_out[H:])
    bo = jnp.zeros((1, 128), jnp.float32).at[0, :NCLS].set(b_out)

    h0, tx = _init_step(ann_p, W_red, b_red.reshape(1, H), W_et, b_et2)
    h = h0
    for _ in range(T - 1):
        a2 = _sc_edge_step()(tx.reshape(NE * NP, H), idx_comb)
        h, tx = _fused_step(a2, h, W_ih, b_ih2, W_hh, b_hh2, W_et, b_et2)
    a2 = _sc_edge_step()(tx.reshape(NE * NP, H), idx_comb)
    h = _gru_step(a2, h, W_ih, b_ih2, W_hh, b_hh2)

    logp_full, pred_full = _pool(h, h0, wgh, wgh0, bg, woh, woh0, bo)
    logp2 = logp_full[0, :NCLS]
    loss = -jnp.take(logp2, labels.astype(jnp.int32)).mean()
    preds = pred_full[0, :1]
    return (loss, preds)


# speedup vs baseline: 1.1790x; 1.0115x over previous
import functools

import jax
import jax.numpy as jnp
from jax import lax
from jax.experimental import pallas as pl
from jax.experimental.pallas import tpu as pltpu
from jax.experimental.pallas import tpu_sc as plsc

N = 10000
NP = 10240
H = 128
NE = 4
T = 5
E = 320000
NCLS = 2

NWORK = 32
CHUNK = 128
NCH = 80
EPW = NCH * CHUNK
EPAD = NWORK * EPW
ACC_ROWS = N + 40

BLK = 1024
NBLK = NP // BLK


def _mm_bias_body(x_ref, w_ref, b_ref, o_ref):
    o_ref[...] = jnp.dot(x_ref[...], w_ref[...],
                         preferred_element_type=jnp.float32) + b_ref[...]


def _emit_pre(h, wet_ref, bet_ref, tx_ref):
    for e in range(NE):
        tx_ref[e] = (jnp.dot(h, wet_ref[e], preferred_element_type=jnp.float32)
                     + bet_ref[:, e * H:(e + 1) * H])


def _init_body(x_ref, wred_ref, bred_ref, wet_ref, bet_ref, h0_ref, tx_ref):
    h0 = jnp.dot(x_ref[...], wred_ref[...],
                 preferred_element_type=jnp.float32) + bred_ref[...]
    h0_ref[...] = h0
    _emit_pre(h0, wet_ref, bet_ref, tx_ref)


def _emit_gru(a_ref, h_ref, wih_ref, bih_ref, whh_ref, bhh_ref):
    a = a_ref[0] + a_ref[1]
    gi = jnp.dot(a, wih_ref[...], preferred_element_type=jnp.float32) + bih_ref[...]
    h = h_ref[...]
    gh = jnp.dot(h, whh_ref[...], preferred_element_type=jnp.float32) + bhh_ref[...]
    r = jax.nn.sigmoid(gi[:, 0:H] + gh[:, 0:H])
    z = jax.nn.sigmoid(gi[:, H:2 * H] + gh[:, H:2 * H])
    n = jnp.tanh(gi[:, 2 * H:3 * H] + r * gh[:, 2 * H:3 * H])
    return (1.0 - z) * n + z * h


def _fused_step_body(a_ref, h_ref, wih_ref, bih_ref, whh_ref, bhh_ref,
                     wet_ref, bet_ref, hn_ref, tx_ref):
    hn = _emit_gru(a_ref, h_ref, wih_ref, bih_ref, whh_ref, bhh_ref)
    hn_ref[...] = hn
    _emit_pre(hn, wet_ref, bet_ref, tx_ref)


def _gru_body(a_ref, h_ref, wih_ref, bih_ref, whh_ref, bhh_ref, o_ref):
    o_ref[...] = _emit_gru(a_ref, h_ref, wih_ref, bih_ref, whh_ref, bhh_ref)


def _pool_body(h_ref, h0_ref, wgh_ref, wgh0_ref, bg_ref, woh_ref, woh0_ref,
               bo_ref, logp_ref, pred_ref):
    h = h_ref[...]
    h0 = h0_ref[...]
    g = (jnp.sum(h * wgh_ref[...], axis=1, keepdims=True)
         + jnp.sum(h0 * wgh0_ref[...], axis=1, keepdims=True) + bg_ref[0, 0])
    rows = lax.broadcasted_iota(jnp.int32, (NP, 1), 0)
    valid = rows < N
    g = jnp.where(valid, g, -jnp.inf)
    m = jnp.max(g)
    e = jnp.where(valid, jnp.exp(g - m), 0.0)
    ssum = jnp.sum(e)
    rh = jnp.sum(e * h, axis=0, keepdims=True) / ssum
    rh0 = jnp.sum(e * h0, axis=0, keepdims=True) / ssum
    logits = (jnp.dot(rh, woh_ref[...], preferred_element_type=jnp.float32)
              + jnp.dot(rh0, woh0_ref[...], preferred_element_type=jnp.float32)
              + bo_ref[...])
    lanes = lax.broadcasted_iota(jnp.int32, (1, 128), 1)
    lmask = lanes < NCLS
    lm = jnp.where(lmask, logits, -jnp.inf)
    m2 = jnp.max(lm)
    lse = m2 + jnp.log(jnp.sum(jnp.where(lmask, jnp.exp(lm - m2), 0.0)))
    logp_ref[...] = jnp.where(lmask, logits - lse, 0.0)
    pred = jnp.where(logits[0, 1] > logits[0, 0], 1, 0).astype(jnp.int32)
    pred_ref[...] = jnp.where(lanes == 0, pred, 0)


def _sc_body(tx_hbm, idx_hbm, out_hbm, ibuf, rows0, rows1, rows2, acc,
             sem0, sem1, sem2, isem0, isem1, isem2, isem3, isem4, isem5):
    c = lax.axis_index("c")
    s = lax.axis_index("s")
    w = c * 16 + s
    rows = (rows0, rows1, rows2)
    sems = (sem0, sem1, sem2)
    isems = (isem0, isem1, isem2, isem3, isem4, isem5)
    base = w * NCH

    for q in range(3):
        pltpu.sync_copy(idx_hbm.at[base + q], ibuf.at[q])
    for q in range(3, 6):
        pltpu.async_copy(idx_hbm.at[base + q], ibuf.at[q], isems[q])
    for q in (1, 2):
        pltpu.async_copy(tx_hbm.at[ibuf.at[q, 0]], rows[q], sems[q])

    zero16 = jnp.zeros((16,), jnp.float32)

    def _zrow(i, carry):
        for j in range(8):
            rows0[i, pl.ds(j * 16, 16)] = zero16
        return carry

    lax.fori_loop(0, CHUNK, _zrow, 0)
    for k in range(4):
        pltpu.sync_copy(rows0, acc.at[pl.ds(s * 627 + k * CHUNK, CHUNK)])

    @pl.when(s == 15)
    def _zero_tail():
        pltpu.sync_copy(rows0.at[pl.ds(0, 123)], acc.at[pl.ds(9917, 123)])

    @pl.when(s < 15)
    def _zero_mid():
        pltpu.sync_copy(rows0.at[pl.ds(0, 115)],
                        acc.at[pl.ds(s * 627 + 512, 115)])

    pltpu.async_copy(tx_hbm.at[ibuf.at[0, 0]], rows0, sem0)
    plsc.subcore_barrier()

    def _step(j, jj):
        q3, q6 = j % 3, j % 6
        if j + 3 < NCH:
            pltpu.make_async_copy(idx_hbm.at[base], ibuf.at[(j + 3) % 6],
                                  isems[(j + 3) % 6]).wait()
        pltpu.make_async_copy(tx_hbm.at[pl.ds(0, CHUNK)], rows[q3], sems[q3]).wait()
        pltpu.sync_copy(rows[q3], acc.at[ibuf.at[q6, 1]], add=True)
        if j + 3 < NCH:
            pltpu.async_copy(tx_hbm.at[ibuf.at[(j + 3) % 6, 0]], rows[q3], sems[q3])
        if j + 6 < NCH:
            pltpu.async_copy(idx_hbm.at[jj + 6], ibuf.at[q6], isems[q6])

    def _hex(i, carry):
        j0 = i * 6
        for b in range(6):
            _step(b, base + j0 + b)
        return carry

    NMAIN = ((NCH - 12) // 6) * 6
    lax.fori_loop(0, NMAIN // 6, _hex, 0)
    for j in range(NMAIN, NCH):
        _step(j, base + j)
    plsc.subcore_barrier()

    @pl.when(s < 15)
    def _write_mid():
        for k in range(5):
            r0 = s * 640 + k * CHUNK
            pltpu.sync_copy(acc.at[pl.ds(r0, CHUNK)], out_hbm.at[c, pl.ds(r0, CHUNK)])

    @pl.when(s == 15)
    def _write_tail():
        for dst_off, src_off, n in ((9600, 9600, 128), (9728, 9728, 128),
                                    (9856, 9856, 128), (9984, 9984, 16),
                                    (10000, 0, 128), (10128, 0, 112)):
            pltpu.sync_copy(acc.at[pl.ds(src_off, n)],
                            out_hbm.at[c, pl.ds(dst_off, n)])


@functools.cache
def _sc_edge_step():
    mesh = plsc.VectorSubcoreMesh(core_axis_name="c", subcore_axis_name="s",
                                  num_cores=2, num_subcores=16)
    return pl.kernel(
        _sc_body,
        out_type=jax.ShapeDtypeStruct((2, NP, H), jnp.float32),
        mesh=mesh,
        scratch_types=(
            [pltpu.VMEM((6, 2, CHUNK), jnp.int32)]
            + [pltpu.VMEM((CHUNK, H), jnp.float32) for _ in range(3)]
            + [pltpu.VMEM_SHARED((ACC_ROWS, H), jnp.float32)]
            + [pltpu.SemaphoreType.DMA] * 9
        ),
    )


def _init_step(x, W_red, b_red2, W_et, b_et2):
    return pl.pallas_call(
        _init_body,
        grid=(NBLK,),
        in_specs=[pl.BlockSpec((BLK, H), lambda i: (i, 0)),
                  pl.BlockSpec((H, H), lambda i: (0, 0)),
                  pl.BlockSpec((1, H), lambda i: (0, 0)),
                  pl.BlockSpec((NE, H, H), lambda i: (0, 0, 0)),
                  pl.BlockSpec((1, NE * H), lambda i: (0, 0))],
        out_specs=[pl.BlockSpec((BLK, H), lambda i: (i, 0)),
                   pl.BlockSpec((NE, BLK, H), lambda i: (0, i, 0))],
        out_shape=[jax.ShapeDtypeStruct((NP, H), jnp.float32),
                   jax.ShapeDtypeStruct((NE, NP, H), jnp.float32)],
    )(x, W_red, b_red2, W_et, b_et2)


def _fused_step(a2, h, W_ih, b_ih2, W_hh, b_hh2, W_et, b_et2):
    return pl.pallas_call(
        _fused_step_body,
        grid=(NBLK,),
        in_specs=[pl.BlockSpec((2, BLK, H), lambda i: (0, i, 0)),
                  pl.BlockSpec((BLK, H), lambda i: (i, 0)),
                  pl.BlockSpec((H, 3 * H), lambda i: (0, 0)),
                  pl.BlockSpec((1, 3 * H), lambda i: (0, 0)),
                  pl.BlockSpec((H, 3 * H), lambda i: (0, 0)),
                  pl.BlockSpec((1, 3 * H), lambda i: (0, 0)),
                  pl.BlockSpec((NE, H, H), lambda i: (0, 0, 0)),
                  pl.BlockSpec((1, NE * H), lambda i: (0, 0))],
        out_specs=[pl.BlockSpec((BLK, H), lambda i: (i, 0)),
                   pl.BlockSpec((NE, BLK, H), lambda i: (0, i, 0))],
        out_shape=[jax.ShapeDtypeStruct((NP, H), jnp.float32),
                   jax.ShapeDtypeStruct((NE, NP, H), jnp.float32)],
    )(a2, h, W_ih, b_ih2, W_hh, b_hh2, W_et, b_et2)


def _gru_step(a2, h, W_ih, b_ih2, W_hh, b_hh2):
    return pl.pallas_call(
        _gru_body,
        grid=(NBLK,),
        in_specs=[pl.BlockSpec((2, BLK, H), lambda i: (0, i, 0)),
                  pl.BlockSpec((BLK, H), lambda i: (i, 0)),
                  pl.BlockSpec((H, 3 * H), lambda i: (0, 0)),
                  pl.BlockSpec((1, 3 * H), lambda i: (0, 0)),
                  pl.BlockSpec((H, 3 * H), lambda i: (0, 0)),
                  pl.BlockSpec((1, 3 * H), lambda i: (0, 0))],
        out_specs=pl.BlockSpec((BLK, H), lambda i: (i, 0)),
        out_shape=jax.ShapeDtypeStruct((NP, H), jnp.float32),
    )(a2, h, W_ih, b_ih2, W_hh, b_hh2)


def _pool(h, h0, wgh, wgh0, bg, woh, woh0, bo):
    return pl.pallas_call(
        _pool_body,
        in_specs=[pl.BlockSpec((NP, H), lambda: (0, 0)),
                  pl.BlockSpec((NP, H), lambda: (0, 0)),
                  pl.BlockSpec((1, H), lambda: (0, 0)),
                  pl.BlockSpec((1, H), lambda: (0, 0)),
                  pl.BlockSpec((1, 1), lambda: (0, 0)),
                  pl.BlockSpec((H, 128), lambda: (0, 0)),
                  pl.BlockSpec((H, 128), lambda: (0, 0)),
                  pl.BlockSpec((1, 128), lambda: (0, 0))],
        out_specs=[pl.BlockSpec((1, 128), lambda: (0, 0)),
                   pl.BlockSpec((1, 128), lambda: (0, 0))],
        out_shape=[jax.ShapeDtypeStruct((1, 128), jnp.float32),
                   jax.ShapeDtypeStruct((1, 128), jnp.int32)],
    )(h, h0, wgh, wgh0, bg, woh, woh0, bo)


def kernel(annotation, edge_index, etypes, labels, W_red, b_red, W_et, b_et,
           W_ih, b_ih, W_hh, b_hh, W_gate, b_gate, W_out, b_out):
    src = edge_index[0].astype(jnp.int32)
    dst = edge_index[1].astype(jnp.int32)
    et = etypes.astype(jnp.int32)
    gidx = et * NP + src
    npad = EPAD - E
    pad_g = jnp.arange(npad, dtype=jnp.int32) % 1024
    pad_d = N + jnp.arange(npad, dtype=jnp.int32) % 40
    gidx2 = jnp.concatenate([gidx, pad_g]).reshape(EPAD // CHUNK, CHUNK)
    dst2 = jnp.concatenate([dst, pad_d]).reshape(EPAD // CHUNK, CHUNK)
    idx_comb = jnp.stack([gidx2, dst2], axis=1)

    ann_p = jnp.pad(annotation.astype(jnp.float32), ((0, NP - N), (0, 0)))
    b_et2 = b_et.reshape(1, NE * H)
    b_hh2 = b_hh.reshape(1, 3 * H)
    b_ih2 = b_ih.reshape(1, 3 * H)
    wgh = W_gate[:H].reshape(1, H)
    wgh0 = W_gate[H:].reshape(1, H)
    bg = b_gate.reshape(1, 1)
    woh = jnp.zeros((H, 128), jnp.float32).at[:, :NCLS].set(W_out[:H])
    woh0 = jnp.zeros((H, 128), jnp.float32).at[:, :NCLS].set(W_out[H:])
    bo = jnp.zeros((1, 128), jnp.float32).at[0, :NCLS].set(b_out)

    h0, tx = _init_step(ann_p, W_red, b_red.reshape(1, H), W_et, b_et2)
    h = h0
    for _ in range(T - 1):
        a2 = _sc_edge_step()(tx.reshape(NE * NP, H), idx_comb)
        h, tx = _fused_step(a2, h, W_ih, b_ih2, W_hh, b_hh2, W_et, b_et2)
    a2 = _sc_edge_step()(tx.reshape(NE * NP, H), idx_comb)
    h = _gru_step(a2, h, W_ih, b_ih2, W_hh, b_hh2)

    logp_full, pred_full = _pool(h, h0, wgh, wgh0, bg, woh, woh0, bo)
    logp2 = logp_full[0, :NCLS]
    loss = -jnp.take(logp2, labels.astype(jnp.int32)).mean()
    preds = pred_full[0, :1]
    return (loss, preds)


# speedup vs baseline: 1.1878x; 1.0075x over previous
import functools

import jax
import jax.numpy as jnp
from jax import lax
from jax.experimental import pallas as pl
from jax.experimental.pallas import tpu as pltpu
from jax.experimental.pallas import tpu_sc as plsc

N = 10000
NP = 10240
H = 128
NE = 4
T = 5
E = 320000
NCLS = 2

NWORK = 32
CHUNK = 128
NCH = 80
EPW = NCH * CHUNK
EPAD = NWORK * EPW
ACC_ROWS = N + 40

BLK = 1024
NBLK = NP // BLK


def _mm_bias_body(x_ref, w_ref, b_ref, o_ref):
    o_ref[...] = jnp.dot(x_ref[...], w_ref[...],
                         preferred_element_type=jnp.float32) + b_ref[...]


def _emit_pre(h, wet_ref, bet_ref, tx_ref):
    m = jnp.dot(h, wet_ref[...], preferred_element_type=jnp.float32) + bet_ref[...]
    for e in range(NE):
        tx_ref[e] = m[:, e * H:(e + 1) * H]


def _init_body(x_ref, wred_ref, bred_ref, wet_ref, bet_ref, h0_ref, tx_ref):
    h0 = jnp.dot(x_ref[...], wred_ref[...],
                 preferred_element_type=jnp.float32) + bred_ref[...]
    h0_ref[...] = h0
    _emit_pre(h0, wet_ref, bet_ref, tx_ref)


def _emit_gru(a_ref, h_ref, wih_ref, bih_ref, whh_ref, bhh_ref):
    a = a_ref[0] + a_ref[1]
    gi = jnp.dot(a, wih_ref[...], preferred_element_type=jnp.float32) + bih_ref[...]
    h = h_ref[...]
    gh = jnp.dot(h, whh_ref[...], preferred_element_type=jnp.float32) + bhh_ref[...]
    r = jax.nn.sigmoid(gi[:, 0:H] + gh[:, 0:H])
    z = jax.nn.sigmoid(gi[:, H:2 * H] + gh[:, H:2 * H])
    n = jnp.tanh(gi[:, 2 * H:3 * H] + r * gh[:, 2 * H:3 * H])
    return (1.0 - z) * n + z * h


def _fused_step_body(a_ref, h_ref, wih_ref, bih_ref, whh_ref, bhh_ref,
                     wet_ref, bet_ref, hn_ref, tx_ref):
    hn = _emit_gru(a_ref, h_ref, wih_ref, bih_ref, whh_ref, bhh_ref)
    hn_ref[...] = hn
    _emit_pre(hn, wet_ref, bet_ref, tx_ref)


def _gru_body(a_ref, h_ref, wih_ref, bih_ref, whh_ref, bhh_ref, o_ref):
    o_ref[...] = _emit_gru(a_ref, h_ref, wih_ref, bih_ref, whh_ref, bhh_ref)


def _pool_body(h_ref, h0_ref, wgh_ref, wgh0_ref, bg_ref, woh_ref, woh0_ref,
               bo_ref, logp_ref, pred_ref):
    h = h_ref[...]
    h0 = h0_ref[...]
    g = (jnp.sum(h * wgh_ref[...], axis=1, keepdims=True)
         + jnp.sum(h0 * wgh0_ref[...], axis=1, keepdims=True) + bg_ref[0, 0])
    rows = lax.broadcasted_iota(jnp.int32, (NP, 1), 0)
    valid = rows < N
    g = jnp.where(valid, g, -jnp.inf)
    m = jnp.max(g)
    e = jnp.where(valid, jnp.exp(g - m), 0.0)
    ssum = jnp.sum(e)
    rh = jnp.sum(e * h, axis=0, keepdims=True) / ssum
    rh0 = jnp.sum(e * h0, axis=0, keepdims=True) / ssum
    logits = (jnp.dot(rh, woh_ref[...], preferred_element_type=jnp.float32)
              + jnp.dot(rh0, woh0_ref[...], preferred_element_type=jnp.float32)
              + bo_ref[...])
    lanes = lax.broadcasted_iota(jnp.int32, (1, 128), 1)
    lmask = lanes < NCLS
    lm = jnp.where(lmask, logits, -jnp.inf)
    m2 = jnp.max(lm)
    lse = m2 + jnp.log(jnp.sum(jnp.where(lmask, jnp.exp(lm - m2), 0.0)))
    logp_ref[...] = jnp.where(lmask, logits - lse, 0.0)
    pred = jnp.where(logits[0, 1] > logits[0, 0], 1, 0).astype(jnp.int32)
    pred_ref[...] = jnp.where(lanes == 0, pred, 0)


def _sc_body(tx_hbm, idx_hbm, out_hbm, ibuf, rows0, rows1, rows2, acc,
             sem0, sem1, sem2, isem0, isem1, isem2, isem3, isem4, isem5):
    c = lax.axis_index("c")
    s = lax.axis_index("s")
    w = c * 16 + s
    rows = (rows0, rows1, rows2)
    sems = (sem0, sem1, sem2)
    isems = (isem0, isem1, isem2, isem3, isem4, isem5)
    base = w * NCH

    for q in range(3):
        pltpu.sync_copy(idx_hbm.at[base + q], ibuf.at[q])
    for q in range(3, 6):
        pltpu.async_copy(idx_hbm.at[base + q], ibuf.at[q], isems[q])
    for q in (1, 2):
        pltpu.async_copy(tx_hbm.at[ibuf.at[q, 0]], rows[q], sems[q])

    zero16 = jnp.zeros((16,), jnp.float32)

    def _zrow(i, carry):
        for j in range(8):
            rows0[i, pl.ds(j * 16, 16)] = zero16
        return carry

    lax.fori_loop(0, CHUNK, _zrow, 0)
    for k in range(4):
        pltpu.sync_copy(rows0, acc.at[pl.ds(s * 627 + k * CHUNK, CHUNK)])

    @pl.when(s == 15)
    def _zero_tail():
        pltpu.sync_copy(rows0.at[pl.ds(0, 123)], acc.at[pl.ds(9917, 123)])

    @pl.when(s < 15)
    def _zero_mid():
        pltpu.sync_copy(rows0.at[pl.ds(0, 115)],
                        acc.at[pl.ds(s * 627 + 512, 115)])

    pltpu.async_copy(tx_hbm.at[ibuf.at[0, 0]], rows0, sem0)
    plsc.subcore_barrier()

    def _step(j, jj):
        q3, q6 = j % 3, j % 6
        if j + 3 < NCH:
            pltpu.make_async_copy(idx_hbm.at[base], ibuf.at[(j + 3) % 6],
                                  isems[(j + 3) % 6]).wait()
        pltpu.make_async_copy(tx_hbm.at[pl.ds(0, CHUNK)], rows[q3], sems[q3]).wait()
        pltpu.sync_copy(rows[q3], acc.at[ibuf.at[q6, 1]], add=True)
        if j + 3 < NCH:
            pltpu.async_copy(tx_hbm.at[ibuf.at[(j + 3) % 6, 0]], rows[q3], sems[q3])
        if j + 6 < NCH:
            pltpu.async_copy(idx_hbm.at[jj + 6], ibuf.at[q6], isems[q6])

    def _hex(i, carry):
        j0 = i * 6
        for b in range(6):
            _step(b, base + j0 + b)
        return carry

    NMAIN = ((NCH - 12) // 6) * 6
    lax.fori_loop(0, NMAIN // 6, _hex, 0)
    for j in range(NMAIN, NCH):
        _step(j, base + j)
    plsc.subcore_barrier()

    @pl.when(s < 15)
    def _write_mid():
        for k in range(5):
            r0 = s * 640 + k * CHUNK
            pltpu.sync_copy(acc.at[pl.ds(r0, CHUNK)], out_hbm.at[c, pl.ds(r0, CHUNK)])

    @pl.when(s == 15)
    def _write_tail():
        for dst_off, src_off, n in ((9600, 9600, 128), (9728, 9728, 128),
                                    (9856, 9856, 128), (9984, 9984, 16),
                                    (10000, 0, 128), (10128, 0, 112)):
            pltpu.sync_copy(acc.at[pl.ds(src_off, n)],
                            out_hbm.at[c, pl.ds(dst_off, n)])


@functools.cache
def _sc_edge_step():
    mesh = plsc.VectorSubcoreMesh(core_axis_name="c", subcore_axis_name="s",
                                  num_cores=2, num_subcores=16)
    return pl.kernel(
        _sc_body,
        out_type=jax.ShapeDtypeStruct((2, NP, H), jnp.float32),
        mesh=mesh,
        scratch_types=(
            [pltpu.VMEM((6, 2, CHUNK), jnp.int32)]
            + [pltpu.VMEM((CHUNK, H), jnp.float32) for _ in range(3)]
            + [pltpu.VMEM_SHARED((ACC_ROWS, H), jnp.float32)]
            + [pltpu.SemaphoreType.DMA] * 9
        ),
    )


def _init_step(x, W_red, b_red2, W_et, b_et2):
    return pl.pallas_call(
        _init_body,
        grid=(NBLK,),
        in_specs=[pl.BlockSpec((BLK, H), lambda i: (i, 0)),
                  pl.BlockSpec((H, H), lambda i: (0, 0)),
                  pl.BlockSpec((1, H), lambda i: (0, 0)),
                  pl.BlockSpec((H, NE * H), lambda i: (0, 0)),
                  pl.BlockSpec((1, NE * H), lambda i: (0, 0))],
        out_specs=[pl.BlockSpec((BLK, H), lambda i: (i, 0)),
                   pl.BlockSpec((NE, BLK, H), lambda i: (0, i, 0))],
        out_shape=[jax.ShapeDtypeStruct((NP, H), jnp.float32),
                   jax.ShapeDtypeStruct((NE, NP, H), jnp.float32)],
    )(x, W_red, b_red2, W_et, b_et2)


def _fused_step(a2, h, W_ih, b_ih2, W_hh, b_hh2, W_et, b_et2):
    return pl.pallas_call(
        _fused_step_body,
        grid=(NBLK,),
        in_specs=[pl.BlockSpec((2, BLK, H), lambda i: (0, i, 0)),
                  pl.BlockSpec((BLK, H), lambda i: (i, 0)),
                  pl.BlockSpec((H, 3 * H), lambda i: (0, 0)),
                  pl.BlockSpec((1, 3 * H), lambda i: (0, 0)),
                  pl.BlockSpec((H, 3 * H), lambda i: (0, 0)),
                  pl.BlockSpec((1, 3 * H), lambda i: (0, 0)),
                  pl.BlockSpec((H, NE * H), lambda i: (0, 0)),
                  pl.BlockSpec((1, NE * H), lambda i: (0, 0))],
        out_specs=[pl.BlockSpec((BLK, H), lambda i: (i, 0)),
                   pl.BlockSpec((NE, BLK, H), lambda i: (0, i, 0))],
        out_shape=[jax.ShapeDtypeStruct((NP, H), jnp.float32),
                   jax.ShapeDtypeStruct((NE, NP, H), jnp.float32)],
    )(a2, h, W_ih, b_ih2, W_hh, b_hh2, W_et, b_et2)


def _gru_step(a2, h, W_ih, b_ih2, W_hh, b_hh2):
    return pl.pallas_call(
        _gru_body,
        grid=(NBLK,),
        in_specs=[pl.BlockSpec((2, BLK, H), lambda i: (0, i, 0)),
                  pl.BlockSpec((BLK, H), lambda i: (i, 0)),
                  pl.BlockSpec((H, 3 * H), lambda i: (0, 0)),
                  pl.BlockSpec((1, 3 * H), lambda i: (0, 0)),
                  pl.BlockSpec((H, 3 * H), lambda i: (0, 0)),
                  pl.BlockSpec((1, 3 * H), lambda i: (0, 0))],
        out_specs=pl.BlockSpec((BLK, H), lambda i: (i, 0)),
        out_shape=jax.ShapeDtypeStruct((NP, H), jnp.float32),
    )(a2, h, W_ih, b_ih2, W_hh, b_hh2)


def _pool(h, h0, wgh, wgh0, bg, woh, woh0, bo):
    return pl.pallas_call(
        _pool_body,
        in_specs=[pl.BlockSpec((NP, H), lambda: (0, 0)),
                  pl.BlockSpec((NP, H), lambda: (0, 0)),
                  pl.BlockSpec((1, H), lambda: (0, 0)),
                  pl.BlockSpec((1, H), lambda: (0, 0)),
                  pl.BlockSpec((1, 1), lambda: (0, 0)),
                  pl.BlockSpec((H, 128), lambda: (0, 0)),
                  pl.BlockSpec((H, 128), lambda: (0, 0)),
                  pl.BlockSpec((1, 128), lambda: (0, 0))],
        out_specs=[pl.BlockSpec((1, 128), lambda: (0, 0)),
                   pl.BlockSpec((1, 128), lambda: (0, 0))],
        out_shape=[jax.ShapeDtypeStruct((1, 128), jnp.float32),
                   jax.ShapeDtypeStruct((1, 128), jnp.int32)],
    )(h, h0, wgh, wgh0, bg, woh, woh0, bo)


def kernel(annotation, edge_index, etypes, labels, W_red, b_red, W_et, b_et,
           W_ih, b_ih, W_hh, b_hh, W_gate, b_gate, W_out, b_out):
    src = edge_index[0].astype(jnp.int32)
    dst = edge_index[1].astype(jnp.int32)
    et = etypes.astype(jnp.int32)
    gidx = et * NP + src
    npad = EPAD - E
    pad_g = jnp.arange(npad, dtype=jnp.int32) % 1024
    pad_d = N + jnp.arange(npad, dtype=jnp.int32) % 40
    gidx2 = jnp.concatenate([gidx, pad_g]).reshape(EPAD // CHUNK, CHUNK)
    dst2 = jnp.concatenate([dst, pad_d]).reshape(EPAD // CHUNK, CHUNK)
    idx_comb = jnp.stack([gidx2, dst2], axis=1)

    ann_p = jnp.pad(annotation.astype(jnp.float32), ((0, NP - N), (0, 0)))
    W_cat = jnp.transpose(W_et, (1, 0, 2)).reshape(H, NE * H)
    b_et2 = b_et.reshape(1, NE * H)
    b_hh2 = b_hh.reshape(1, 3 * H)
    b_ih2 = b_ih.reshape(1, 3 * H)
    wgh = W_gate[:H].reshape(1, H)
    wgh0 = W_gate[H:].reshape(1, H)
    bg = b_gate.reshape(1, 1)
    woh = jnp.zeros((H, 128), jnp.float32).at[:, :NCLS].set(W_out[:H])
    woh0 = jnp.zeros((H, 128), jnp.float32).at[:, :NCLS].set(W_out[H:])
    bo = jnp.zeros((1, 128), jnp.float32).at[0, :NCLS].set(b_out)

    h0, tx = _init_step(ann_p, W_red, b_red.reshape(1, H), W_cat, b_et2)
    h = h0
    for _ in range(T - 1):
        a2 = _sc_edge_step()(tx.reshape(NE * NP, H), idx_comb)
        h, tx = _fused_step(a2, h, W_ih, b_ih2, W_hh, b_hh2, W_cat, b_et2)
    a2 = _sc_edge_step()(tx.reshape(NE * NP, H), idx_comb)
    h = _gru_step(a2, h, W_ih, b_ih2, W_hh, b_hh2)

    logp_full, pred_full = _pool(h, h0, wgh, wgh0, bg, woh, woh0, bo)
    logp2 = logp_full[0, :NCLS]
    loss = -jnp.take(logp2, labels.astype(jnp.int32)).mean()
    preds = pred_full[0, :1]
    return (loss, preds)


# speedup vs baseline: 1.1890x; 1.0010x over previous
import functools

import jax
import jax.numpy as jnp
from jax import lax
from jax.experimental import pallas as pl
from jax.experimental.pallas import tpu as pltpu
from jax.experimental.pallas import tpu_sc as plsc

N = 10000
NP = 10240
H = 128
NE = 4
T = 5
E = 320000
NCLS = 2

NWORK = 32
CHUNK = 128
NCH = 80
EPW = NCH * CHUNK
EPAD = NWORK * EPW
ACC_ROWS = N + 40

BLK = 1024
NBLK = NP // BLK


def _emit_pre(h, wet_ref, bet_ref, tx_ref):
    m = jnp.dot(h, wet_ref[...], preferred_element_type=jnp.float32) + bet_ref[...]
    for e in range(NE):
        tx_ref[e] = m[:, e * H:(e + 1) * H]


def _init_body(x_ref, wred_ref, bred_ref, wet_ref, bet_ref, h0_ref, tx_ref):
    h0 = jnp.dot(x_ref[...], wred_ref[...],
                 preferred_element_type=jnp.float32) + bred_ref[...]
    h0_ref[...] = h0
    _emit_pre(h0, wet_ref, bet_ref, tx_ref)


def _emit_gru(a_ref, h_ref, wih_ref, bih_ref, whh_ref, bhh_ref):
    a = a_ref[0] + a_ref[1]
    gi = jnp.dot(a, wih_ref[...], preferred_element_type=jnp.float32) + bih_ref[...]
    h = h_ref[...]
    gh = jnp.dot(h, whh_ref[...], preferred_element_type=jnp.float32) + bhh_ref[...]
    r = jax.nn.sigmoid(gi[:, 0:H] + gh[:, 0:H])
    z = jax.nn.sigmoid(gi[:, H:2 * H] + gh[:, H:2 * H])
    n = jnp.tanh(gi[:, 2 * H:3 * H] + r * gh[:, 2 * H:3 * H])
    return (1.0 - z) * n + z * h


def _fused_step_body(a_ref, h_ref, wih_ref, bih_ref, whh_ref, bhh_ref,
                     wet_ref, bet_ref, hn_ref, tx_ref):
    hn = _emit_gru(a_ref, h_ref, wih_ref, bih_ref, whh_ref, bhh_ref)
    hn_ref[...] = hn
    _emit_pre(hn, wet_ref, bet_ref, tx_ref)


def _gru_body(a_ref, h_ref, wih_ref, bih_ref, whh_ref, bhh_ref, o_ref):
    o_ref[...] = _emit_gru(a_ref, h_ref, wih_ref, bih_ref, whh_ref, bhh_ref)


def _pool_body(h_ref, h0_ref, wgh_ref, wgh0_ref, bg_ref, woh_ref, woh0_ref,
               bo_ref, logp_ref, pred_ref):
    h = h_ref[...]
    h0 = h0_ref[...]
    g = (jnp.sum(h * wgh_ref[...], axis=1, keepdims=True)
         + jnp.sum(h0 * wgh0_ref[...], axis=1, keepdims=True) + bg_ref[0, 0])
    rows = lax.broadcasted_iota(jnp.int32, (NP, 1), 0)
    valid = rows < N
    g = jnp.where(valid, g, -jnp.inf)
    m = jnp.max(g)
    e = jnp.where(valid, jnp.exp(g - m), 0.0)
    ssum = jnp.sum(e)
    rh = jnp.sum(e * h, axis=0, keepdims=True) / ssum
    rh0 = jnp.sum(e * h0, axis=0, keepdims=True) / ssum
    logits = (jnp.dot(rh, woh_ref[...], preferred_element_type=jnp.float32)
              + jnp.dot(rh0, woh0_ref[...], preferred_element_type=jnp.float32)
              + bo_ref[...])
    lanes = lax.broadcasted_iota(jnp.int32, (1, 128), 1)
    lmask = lanes < NCLS
    lm = jnp.where(lmask, logits, -jnp.inf)
    m2 = jnp.max(lm)
    lse = m2 + jnp.log(jnp.sum(jnp.where(lmask, jnp.exp(lm - m2), 0.0)))
    logp_ref[...] = jnp.where(lmask, logits - lse, 0.0)
    pred = jnp.where(logits[0, 1] > logits[0, 0], 1, 0).astype(jnp.int32)
    pred_ref[...] = jnp.where(lanes == 0, pred, 0)


def _sc_body(tx_hbm, idx_hbm, out_hbm, ibuf, rows0, rows1, rows2, acc,
             sem0, sem1, sem2, isem0, isem1, isem2, isem3, isem4, isem5):
    c = lax.axis_index("c")
    s = lax.axis_index("s")
    w = c * 16 + s
    rows = (rows0, rows1, rows2)
    sems = (sem0, sem1, sem2)
    isems = (isem0, isem1, isem2, isem3, isem4, isem5)
    base = w * NCH

    for q in range(3):
        pltpu.sync_copy(idx_hbm.at[base + q], ibuf.at[q])
    for q in range(3, 6):
        pltpu.async_copy(idx_hbm.at[base + q], ibuf.at[q], isems[q])
    for q in (1, 2):
        pltpu.async_copy(tx_hbm.at[ibuf.at[q, 0]], rows[q], sems[q])

    zero16 = jnp.zeros((16,), jnp.float32)

    def _zrow(i, carry):
        for j in range(8):
            rows0[i, pl.ds(j * 16, 16)] = zero16
        return carry

    lax.fori_loop(0, CHUNK, _zrow, 0)
    for k in range(4):
        pltpu.sync_copy(rows0, acc.at[pl.ds(s * 627 + k * CHUNK, CHUNK)])

    @pl.when(s == 15)
    def _zero_tail():
        pltpu.sync_copy(rows0.at[pl.ds(0, 123)], acc.at[pl.ds(9917, 123)])

    @pl.when(s < 15)
    def _zero_mid():
        pltpu.sync_copy(rows0.at[pl.ds(0, 115)],
                        acc.at[pl.ds(s * 627 + 512, 115)])

    pltpu.async_copy(tx_hbm.at[ibuf.at[0, 0]], rows0, sem0)
    plsc.subcore_barrier()

    def _step(j, jj):
        q3, q6 = j % 3, j % 6
        if j + 3 < NCH:
            pltpu.make_async_copy(idx_hbm.at[base], ibuf.at[(j + 3) % 6],
                                  isems[(j + 3) % 6]).wait()
        pltpu.make_async_copy(tx_hbm.at[pl.ds(0, CHUNK)], rows[q3], sems[q3]).wait()
        pltpu.sync_copy(rows[q3], acc.at[ibuf.at[q6, 1]], add=True)
        if j + 3 < NCH:
            pltpu.async_copy(tx_hbm.at[ibuf.at[(j + 3) % 6, 0]], rows[q3], sems[q3])
        if j + 6 < NCH:
            pltpu.async_copy(idx_hbm.at[jj + 6], ibuf.at[q6], isems[q6])

    def _hex(i, carry):
        j0 = i * 6
        for b in range(6):
            _step(b, base + j0 + b)
        return carry

    NMAIN = ((NCH - 12) // 6) * 6
    lax.fori_loop(0, NMAIN // 6, _hex, 0)
    for j in range(NMAIN, NCH):
        _step(j, base + j)
    plsc.subcore_barrier()

    @pl.when(s < 15)
    def _write_mid():
        for k in range(5):
            r0 = s * 640 + k * CHUNK
            pltpu.sync_copy(acc.at[pl.ds(r0, CHUNK)], out_hbm.at[c, pl.ds(r0, CHUNK)])

    @pl.when(s == 15)
    def _write_tail():
        for dst_off, src_off, n in ((9600, 9600, 128), (9728, 9728, 128),
                                    (9856, 9856, 128), (9984, 9984, 16),
                                    (10000, 0, 128), (10128, 0, 112)):
            pltpu.sync_copy(acc.at[pl.ds(src_off, n)],
                            out_hbm.at[c, pl.ds(dst_off, n)])


@functools.cache
def _sc_edge_step():
    mesh = plsc.VectorSubcoreMesh(core_axis_name="c", subcore_axis_name="s",
                                  num_cores=2, num_subcores=16)
    return pl.kernel(
        _sc_body,
        out_type=jax.ShapeDtypeStruct((2, NP, H), jnp.float32),
        mesh=mesh,
        scratch_types=(
            [pltpu.VMEM((6, 2, CHUNK), jnp.int32)]
            + [pltpu.VMEM((CHUNK, H), jnp.float32) for _ in range(3)]
            + [pltpu.VMEM_SHARED((ACC_ROWS, H), jnp.float32)]
            + [pltpu.SemaphoreType.DMA] * 9
        ),
    )


def _init_step(x, W_red, b_red2, W_et, b_et2):
    return pl.pallas_call(
        _init_body,
        grid=(NBLK,),
        in_specs=[pl.BlockSpec((BLK, H), lambda i: (i, 0)),
                  pl.BlockSpec((H, H), lambda i: (0, 0)),
                  pl.BlockSpec((1, H), lambda i: (0, 0)),
                  pl.BlockSpec((H, NE * H), lambda i: (0, 0)),
                  pl.BlockSpec((1, NE * H), lambda i: (0, 0))],
        out_specs=[pl.BlockSpec((BLK, H), lambda i: (i, 0)),
                   pl.BlockSpec((NE, BLK, H), lambda i: (0, i, 0))],
        out_shape=[jax.ShapeDtypeStruct((NP, H), jnp.float32),
                   jax.ShapeDtypeStruct((NE, NP, H), jnp.float32)],
    )(x, W_red, b_red2, W_et, b_et2)


def _fused_step(a2, h, W_ih, b_ih2, W_hh, b_hh2, W_et, b_et2):
    return pl.pallas_call(
        _fused_step_body,
        grid=(NBLK,),
        in_specs=[pl.BlockSpec((2, BLK, H), lambda i: (0, i, 0)),
                  pl.BlockSpec((BLK, H), lambda i: (i, 0)),
                  pl.BlockSpec((H, 3 * H), lambda i: (0, 0)),
                  pl.BlockSpec((1, 3 * H), lambda i: (0, 0)),
                  pl.BlockSpec((H, 3 * H), lambda i: (0, 0)),
                  pl.BlockSpec((1, 3 * H), lambda i: (0, 0)),
                  pl.BlockSpec((H, NE * H), lambda i: (0, 0)),
                  pl.BlockSpec((1, NE * H), lambda i: (0, 0))],
        out_specs=[pl.BlockSpec((BLK, H), lambda i: (i, 0)),
                   pl.BlockSpec((NE, BLK, H), lambda i: (0, i, 0))],
        out_shape=[jax.ShapeDtypeStruct((NP, H), jnp.float32),
                   jax.ShapeDtypeStruct((NE, NP, H), jnp.float32)],
    )(a2, h, W_ih, b_ih2, W_hh, b_hh2, W_et, b_et2)


def _gru_step(a2, h, W_ih, b_ih2, W_hh, b_hh2):
    return pl.pallas_call(
        _gru_body,
        grid=(NBLK,),
        in_specs=[pl.BlockSpec((2, BLK, H), lambda i: (0, i, 0)),
                  pl.BlockSpec((BLK, H), lambda i: (i, 0)),
                  pl.BlockSpec((H, 3 * H), lambda i: (0, 0)),
                  pl.BlockSpec((1, 3 * H), lambda i: (0, 0)),
                  pl.BlockSpec((H, 3 * H), lambda i: (0, 0)),
                  pl.BlockSpec((1, 3 * H), lambda i: (0, 0))],
        out_specs=pl.BlockSpec((BLK, H), lambda i: (i, 0)),
        out_shape=jax.ShapeDtypeStruct((NP, H), jnp.float32),
    )(a2, h, W_ih, b_ih2, W_hh, b_hh2)


def _pool(h, h0, wgh, wgh0, bg, woh, woh0, bo):
    return pl.pallas_call(
        _pool_body,
        in_specs=[pl.BlockSpec((NP, H), lambda: (0, 0)),
                  pl.BlockSpec((NP, H), lambda: (0, 0)),
                  pl.BlockSpec((1, H), lambda: (0, 0)),
                  pl.BlockSpec((1, H), lambda: (0, 0)),
                  pl.BlockSpec((1, 1), lambda: (0, 0)),
                  pl.BlockSpec((H, 128), lambda: (0, 0)),
                  pl.BlockSpec((H, 128), lambda: (0, 0)),
                  pl.BlockSpec((1, 128), lambda: (0, 0))],
        out_specs=[pl.BlockSpec((1, 128), lambda: (0, 0)),
                   pl.BlockSpec((1, 128), lambda: (0, 0))],
        out_shape=[jax.ShapeDtypeStruct((1, 128), jnp.float32),
                   jax.ShapeDtypeStruct((1, 128), jnp.int32)],
    )(h, h0, wgh, wgh0, bg, woh, woh0, bo)


def kernel(annotation, edge_index, etypes, labels, W_red, b_red, W_et, b_et,
           W_ih, b_ih, W_hh, b_hh, W_gate, b_gate, W_out, b_out):
    src = edge_index[0].astype(jnp.int32)
    dst = edge_index[1].astype(jnp.int32)
    et = etypes.astype(jnp.int32)
    gidx = et * NP + src
    npad = EPAD - E
    pad_g = jnp.arange(npad, dtype=jnp.int32) % 1024
    pad_d = N + jnp.arange(npad, dtype=jnp.int32) % 40
    gidx2 = jnp.concatenate([gidx, pad_g]).reshape(EPAD // CHUNK, CHUNK)
    dst2 = jnp.concatenate([dst, pad_d]).reshape(EPAD // CHUNK, CHUNK)
    idx_comb = jnp.stack([gidx2, dst2], axis=1)

    ann_p = jnp.pad(annotation.astype(jnp.float32), ((0, NP - N), (0, 0)))
    W_cat = jnp.transpose(W_et, (1, 0, 2)).reshape(H, NE * H)
    b_et2 = b_et.reshape(1, NE * H)
    b_hh2 = b_hh.reshape(1, 3 * H)
    b_ih2 = b_ih.reshape(1, 3 * H)
    wgh = W_gate[:H].reshape(1, H)
    wgh0 = W_gate[H:].reshape(1, H)
    bg = b_gate.reshape(1, 1)
    woh = jnp.zeros((H, 128), jnp.float32).at[:, :NCLS].set(W_out[:H])
    woh0 = jnp.zeros((H, 128), jnp.float32).at[:, :NCLS].set(W_out[H:])
    bo = jnp.zeros((1, 128), jnp.float32).at[0, :NCLS].set(b_out)

    h0, tx = _init_step(ann_p, W_red, b_red.reshape(1, H), W_cat, b_et2)
    h = h0
    for _ in range(T - 1):
        a2 = _sc_edge_step()(tx.reshape(NE * NP, H), idx_comb)
        h, tx = _fused_step(a2, h, W_ih, b_ih2, W_hh, b_hh2, W_cat, b_et2)
    a2 = _sc_edge_step()(tx.reshape(NE * NP, H), idx_comb)
    h = _gru_step(a2, h, W_ih, b_ih2, W_hh, b_hh2)

    logp_full, pred_full = _pool(h, h0, wgh, wgh0, bg, woh, woh0, bo)
    logp2 = logp_full[0, :NCLS]
    loss = -jnp.take(logp2, labels.astype(jnp.int32)).mean()
    preds = pred_full[0, :1]
    return (loss, preds)
